# Initial kernel scaffold; baseline (speedup 1.0000x reference)
import jax
import jax.numpy as jnp
from jax.experimental import pallas as pl


def kernel(x, edge_index, edge_weight, W1, b1, W2, b2):
    raise NotImplementedError("write your pallas kernel here")


# speedup vs baseline: 9.8275x; 9.8275x over previous
import functools
import jax
import jax.numpy as jnp
from jax import lax
from jax.experimental import pallas as pl
from jax.experimental.pallas import tpu as pltpu
from jax.experimental.pallas import tpu_sc as plsc

N = 10000
CH = 128
BLK = 1000
NP = 10112
RTD = 632
NA = 10112
RT = 632

NC = 2
NS = 16
NW = NC * NS
CK = 64
NCH = 160
NCH2 = 80
EP = NCH * CK
E_PAD = NW * EP

_mesh = plsc.VectorSubcoreMesh(
    core_axis_name="c", subcore_axis_name="s", num_cores=NC, num_subcores=NS)


def _stage_a_body(x_ref, w_ref, d0_ref, d1_ref, o_ref):
    deg = d0_ref[:, :1] + d1_ref[:, :1] + 1.0
    dinv = lax.rsqrt(deg)
    o_ref[...] = jnp.dot(x_ref[...], w_ref[...],
                         preferred_element_type=jnp.float32) * dinv


def _stage_b_body(p0_ref, p1_ref, xs_ref, w_ref, b_ref, d0_ref, d1_ref, o_ref):
    deg = d0_ref[:, :1] + d1_ref[:, :1] + 1.0
    dinv = lax.rsqrt(deg)
    h = jnp.maximum(dinv * (p0_ref[...] + p1_ref[...] + xs_ref[...]) + b_ref[...], 0.0)
    o_ref[...] = jnp.dot(h, w_ref[...], preferred_element_type=jnp.float32) * dinv


def _stage_c_body(p0_ref, p1_ref, xs_ref, b_ref, d0_ref, d1_ref, o_ref):
    deg = d0_ref[:, :1] + d1_ref[:, :1] + 1.0
    dinv = lax.rsqrt(deg)
    o_ref[...] = dinv * (p0_ref[...] + p1_ref[...] + xs_ref[...]) + b_ref[...]


def _row_spec():
    return pl.BlockSpec((BLK, CH), lambda i: (i, 0))


def _deg_spec():
    return pl.BlockSpec((BLK, 16), lambda i: (i, 0))


def _stage_a(x, W1, d0, d1):
    return pl.pallas_call(
        _stage_a_body,
        grid=(N // BLK,),
        in_specs=[_row_spec(),
                  pl.BlockSpec((CH, CH), lambda i: (0, 0)),
                  _deg_spec(), _deg_spec()],
        out_specs=_row_spec(),
        out_shape=jax.ShapeDtypeStruct((N, CH), jnp.float32),
    )(x, W1, d0, d1)


def _stage_b(p0, p1, xs, W2, b1, d0, d1):
    return pl.pallas_call(
        _stage_b_body,
        grid=(N // BLK,),
        in_specs=[_row_spec(), _row_spec(), _row_spec(),
                  pl.BlockSpec((CH, CH), lambda i: (0, 0)),
                  pl.BlockSpec((1, CH), lambda i: (0, 0)),
                  _deg_spec(), _deg_spec()],
        out_specs=_row_spec(),
        out_shape=jax.ShapeDtypeStruct((N, CH), jnp.float32),
    )(p0, p1, xs, W2, b1, d0, d1)


def _stage_c(p0, p1, xs, b2, d0, d1):
    return pl.pallas_call(
        _stage_c_body,
        grid=(N // BLK,),
        in_specs=[_row_spec(), _row_spec(), _row_spec(),
                  pl.BlockSpec((1, CH), lambda i: (0, 0)),
                  _deg_spec(), _deg_spec()],
        out_specs=_row_spec(),
        out_shape=jax.ShapeDtypeStruct((N, CH), jnp.float32),
    )(p0, p1, xs, b2, d0, d1)


def _unpack_sd(sdv, srcv, nrows):
    def row(r, _):
        for k in range(CK // 16):
            v = sdv[r, pl.ds(k * 16, 16)]
            if srcv is not None:
                srcv[r, pl.ds(k * 16, 16)] = v & 0xFFFF
            sdv[r, pl.ds(k * 16, 16)] = lax.shift_right_logical(v, 16)
        return 0
    lax.fori_loop(0, nrows, row, 0)


def _deg_body(sd3, ew2, out, sdv, ewv, vbuf, dacc):
    cid = lax.axis_index("c")
    sid = lax.axis_index("s")
    wid = cid * NS + sid

    pltpu.sync_copy(sd3.at[wid], sdv)
    pltpu.sync_copy(ew2.at[wid], ewv)
    _unpack_sd(sdv, None, NCH)

    def zv(r, _):
        vbuf[r, :] = jnp.zeros((16,), jnp.float32)
        return 0
    lax.fori_loop(0, CK, zv, 0)

    r0 = sid * RTD
    for i in range(RTD // CK):
        pltpu.sync_copy(vbuf, dacc.at[pl.ds(r0 + i * CK, CK)])
    pltpu.sync_copy(vbuf.at[pl.ds(0, RTD % CK)],
                    dacc.at[pl.ds(r0 + (RTD // CK) * CK, RTD % CK)])
    plsc.subcore_barrier()

    e0 = jnp.where(lax.iota(jnp.int32, 16) == 0, 1.0, 0.0)

    def chunk(c, _):
        def row(j, _):
            ewb = plsc.load_gather(ewv, [jnp.full((16,), c * CK + j, jnp.int32)])
            vbuf[j, :] = ewb * e0
            return 0
        lax.fori_loop(0, CK, row, 0)
        pltpu.sync_copy(vbuf, dacc.at[sdv.at[c]], add=True)
        return 0

    lax.fori_loop(0, NCH, chunk, 0)
    plsc.subcore_barrier()

    pltpu.sync_copy(dacc.at[pl.ds(r0, RTD)], out.at[cid, pl.ds(r0, RTD)])


@functools.partial(
    pl.kernel,
    out_type=jax.ShapeDtypeStruct((NC, NP, 16), jnp.float32),
    mesh=_mesh,
    scratch_types=[
        pltpu.VMEM((NCH, CK), jnp.int32),
        pltpu.VMEM((EP,), jnp.float32),
        pltpu.VMEM((CK, 16), jnp.float32),
        pltpu.VMEM_SHARED((NP, 16), jnp.float32),
    ],
    compiler_params=pltpu.CompilerParams(needs_layout_passes=False),
)
def _deg_kernel(sd3, ew2, out, sdv, ewv, vbuf, dacc):
    _deg_body(sd3, ew2, out, sdv, ewv, vbuf, dacc)


def _prop_body(xs, sd3, ew2, out, sdv, srcv, ewv, ga, gb, acc, sema, semb):
    cid = lax.axis_index("c")
    sid = lax.axis_index("s")
    wid = cid * NS + sid

    def zv(r, _):
        for k in range(CH // 16):
            ga[r, pl.ds(k * 16, 16)] = jnp.zeros((16,), jnp.float32)
        return 0
    lax.fori_loop(0, CK, zv, 0)
    r0 = sid * RT
    for i in range(RT // CK):
        pltpu.sync_copy(ga, acc.at[pl.ds(r0 + i * CK, CK)])
    pltpu.sync_copy(ga.at[pl.ds(0, RT % CK)],
                    acc.at[pl.ds(r0 + (RT // CK) * CK, RT % CK)])
    plsc.subcore_barrier()

    def fire(c, buf, sem):
        pltpu.async_copy(xs.at[srcv.at[c]], buf, sem)

    def process(c, buf, sem):
        pltpu.make_async_copy(xs.at[srcv.at[0]], buf, sem).wait()

        def scale(j, _):
            ewb = plsc.load_gather(ewv, [jnp.full((16,), c * CK + j, jnp.int32)])
            for k in range(CH // 16):
                buf[j, pl.ds(k * 16, 16)] = buf[j, pl.ds(k * 16, 16)] * ewb
            return 0
        lax.fori_loop(0, CK, scale, 0)
        pltpu.sync_copy(buf, acc.at[sdv.at[c]], add=True)

    for h in range(2):
        pltpu.sync_copy(sd3.at[wid, pl.ds(h * NCH2, NCH2)], sdv)
        pltpu.sync_copy(ew2.at[wid, pl.ds(h * NCH2 * CK, NCH2 * CK)], ewv)
        _unpack_sd(sdv, srcv, NCH2)

        fire(0, ga, sema)
        fire(1, gb, semb)

        def step(t, _):
            process(2 * t, ga, sema)
            fire(2 * t + 2, ga, sema)
            process(2 * t + 1, gb, semb)
            fire(2 * t + 3, gb, semb)
            return 0

        lax.fori_loop(0, NCH2 // 2 - 1, step, 0)
        process(NCH2 - 2, ga, sema)
        process(NCH2 - 1, gb, semb)

    plsc.subcore_barrier()
    pltpu.sync_copy(acc.at[pl.ds(r0, RT)], out.at[cid, pl.ds(r0, RT)])


@functools.partial(
    pl.kernel,
    out_type=jax.ShapeDtypeStruct((NC, NA, CH), jnp.float32),
    mesh=_mesh,
    scratch_types=[
        pltpu.VMEM((NCH2, CK), jnp.int32),
        pltpu.VMEM((NCH2, CK), jnp.int32),
        pltpu.VMEM((NCH2 * CK,), jnp.float32),
        pltpu.VMEM((CK, CH), jnp.float32),
        pltpu.VMEM((CK, CH), jnp.float32),
        pltpu.VMEM_SHARED((NA, CH), jnp.float32),
        pltpu.SemaphoreType.DMA,
        pltpu.SemaphoreType.DMA,
    ],
    compiler_params=pltpu.CompilerParams(needs_layout_passes=False),
)
def _prop_kernel(xs, sd3, ew2, out, sdv, srcv, ewv, ga, gb, acc, sema, semb):
    _prop_body(xs, sd3, ew2, out, sdv, srcv, ewv, ga, gb, acc, sema, semb)


def kernel(x, edge_index, edge_weight, W1, b1, W2, b2):
    src = edge_index[0].astype(jnp.int32)
    dst = edge_index[1].astype(jnp.int32)
    ew = edge_weight.astype(jnp.float32)
    npad = E_PAD - src.shape[0]
    sd = jnp.concatenate([(dst << 16) | src, jnp.zeros((npad,), jnp.int32)])
    sd3 = sd.reshape(NW, NCH, CK)
    ew2 = jnp.concatenate([ew, jnp.zeros((npad,), jnp.float32)]).reshape(NW, EP)
    b1r = b1.reshape(1, CH)
    b2r = b2.reshape(1, CH)

    degp = _deg_kernel(sd3, ew2)
    d0, d1 = degp[0], degp[1]
    xs1 = _stage_a(x, W1, d0, d1)
    p = _prop_kernel(xs1, sd3, ew2)
    xs2 = _stage_b(p[0], p[1], xs1, W2, b1r, d0, d1)
    q = _prop_kernel(xs2, sd3, ew2)
    return _stage_c(q[0], q[1], xs2, b2r, d0, d1)


# speedup vs baseline: 10.4070x; 1.0590x over previous
import functools
import jax
import jax.numpy as jnp
from jax import lax
from jax.experimental import pallas as pl
from jax.experimental.pallas import tpu as pltpu
from jax.experimental.pallas import tpu_sc as plsc

N = 10000
CH = 128
BLK = 1000
NP = 10112
RTD = 632
NA = 10112
RT = 632

NC = 2
NS = 16
NW = NC * NS
CK = 64
NCH = 160
NCH2 = 80
EP = NCH * CK
E_PAD = NW * EP

_mesh = plsc.VectorSubcoreMesh(
    core_axis_name="c", subcore_axis_name="s", num_cores=NC, num_subcores=NS)


def _stage_a_body(x_ref, w_ref, dp_ref, o_ref):
    deg = jnp.sum(dp_ref[...], axis=1)[:, None] + 1.0
    dinv = lax.rsqrt(deg)
    o_ref[...] = jnp.dot(x_ref[...], w_ref[...],
                         preferred_element_type=jnp.float32) * dinv


def _stage_b_body(p0_ref, p1_ref, xs_ref, w_ref, b_ref, dp_ref, o_ref):
    deg = jnp.sum(dp_ref[...], axis=1)[:, None] + 1.0
    dinv = lax.rsqrt(deg)
    h = jnp.maximum(dinv * (p0_ref[...] + p1_ref[...] + xs_ref[...]) + b_ref[...], 0.0)
    o_ref[...] = jnp.dot(h, w_ref[...], preferred_element_type=jnp.float32) * dinv


def _stage_c_body(p0_ref, p1_ref, xs_ref, b_ref, dp_ref, o_ref):
    deg = jnp.sum(dp_ref[...], axis=1)[:, None] + 1.0
    dinv = lax.rsqrt(deg)
    o_ref[...] = dinv * (p0_ref[...] + p1_ref[...] + xs_ref[...]) + b_ref[...]


def _row_spec():
    return pl.BlockSpec((BLK, CH), lambda i: (i, 0))


def _deg_spec():
    return pl.BlockSpec((BLK, NW), lambda i: (i, 0))


def _stage_a(x, W1, dp):
    return pl.pallas_call(
        _stage_a_body,
        grid=(N // BLK,),
        in_specs=[_row_spec(),
                  pl.BlockSpec((CH, CH), lambda i: (0, 0)),
                  _deg_spec()],
        out_specs=_row_spec(),
        out_shape=jax.ShapeDtypeStruct((N, CH), jnp.float32),
    )(x, W1, dp)


def _stage_b(p0, p1, xs, W2, b1, dp):
    return pl.pallas_call(
        _stage_b_body,
        grid=(N // BLK,),
        in_specs=[_row_spec(), _row_spec(), _row_spec(),
                  pl.BlockSpec((CH, CH), lambda i: (0, 0)),
                  pl.BlockSpec((1, CH), lambda i: (0, 0)),
                  _deg_spec()],
        out_specs=_row_spec(),
        out_shape=jax.ShapeDtypeStruct((N, CH), jnp.float32),
    )(p0, p1, xs, W2, b1, dp)


def _stage_c(p0, p1, xs, b2, dp):
    return pl.pallas_call(
        _stage_c_body,
        grid=(N // BLK,),
        in_specs=[_row_spec(), _row_spec(), _row_spec(),
                  pl.BlockSpec((1, CH), lambda i: (0, 0)),
                  _deg_spec()],
        out_specs=_row_spec(),
        out_shape=jax.ShapeDtypeStruct((N, CH), jnp.float32),
    )(p0, p1, xs, b2, dp)


def _unpack_sd(sdv, srcv, nrows):
    def row(r, _):
        for k in range(CK // 16):
            v = sdv[r, pl.ds(k * 16, 16)]
            if srcv is not None:
                srcv[r, pl.ds(k * 16, 16)] = v & 0xFFFF
            sdv[r, pl.ds(k * 16, 16)] = lax.shift_right_logical(v, 16)
        return 0
    lax.fori_loop(0, nrows, row, 0)


def _deg_body(sd3, ew2, out, sdv, ewv, degp):
    cid = lax.axis_index("c")
    sid = lax.axis_index("s")
    wid = cid * NS + sid

    pltpu.sync_copy(sd3.at[wid], sdv)
    pltpu.sync_copy(ew2.at[wid], ewv)
    _unpack_sd(sdv, None, NCH)

    def zz(r, _):
        degp[pl.ds(r * 16, 16)] = jnp.zeros((16,), jnp.float32)
        return 0
    lax.fori_loop(0, NP // 16, zz, 0)

    def row(r, _):
        for kk in range(CK // 16):
            d16 = sdv[r, pl.ds(kk * 16, 16)]
            w16 = ewv[pl.ds(r * CK + kk * 16, 16)]
            plsc.addupdate_scatter(degp, [d16], w16)
        return 0
    lax.fori_loop(0, NCH, row, 0)

    pltpu.sync_copy(degp, out.at[cid, sid])


@functools.partial(
    pl.kernel,
    out_type=jax.ShapeDtypeStruct((NC, NS, NP), jnp.float32),
    mesh=_mesh,
    scratch_types=[
        pltpu.VMEM((NCH, CK), jnp.int32),
        pltpu.VMEM((EP,), jnp.float32),
        pltpu.VMEM((NP,), jnp.float32),
    ],
    compiler_params=pltpu.CompilerParams(needs_layout_passes=False),
)
def _deg_kernel(sd3, ew2, out, sdv, ewv, degp):
    _deg_body(sd3, ew2, out, sdv, ewv, degp)


def _prop_body(xs, sd3, ew2, out, sdv, srcv, ewv, ga, gb, acc, sema, semb):
    cid = lax.axis_index("c")
    sid = lax.axis_index("s")
    wid = cid * NS + sid

    def zv(r, _):
        for k in range(CH // 16):
            ga[r, pl.ds(k * 16, 16)] = jnp.zeros((16,), jnp.float32)
        return 0
    lax.fori_loop(0, CK, zv, 0)
    r0 = sid * RT
    for i in range(RT // CK):
        pltpu.sync_copy(ga, acc.at[pl.ds(r0 + i * CK, CK)])
    pltpu.sync_copy(ga.at[pl.ds(0, RT % CK)],
                    acc.at[pl.ds(r0 + (RT // CK) * CK, RT % CK)])
    plsc.subcore_barrier()

    def fire(c, buf, sem):
        pltpu.async_copy(xs.at[srcv.at[c]], buf, sem)

    def process(c, buf, sem):
        pltpu.make_async_copy(xs.at[srcv.at[0]], buf, sem).wait()

        def scale(j, _):
            ewb = plsc.load_gather(ewv, [jnp.full((16,), c * CK + j, jnp.int32)])
            for k in range(CH // 16):
                buf[j, pl.ds(k * 16, 16)] = buf[j, pl.ds(k * 16, 16)] * ewb
            return 0
        lax.fori_loop(0, CK, scale, 0)
        pltpu.sync_copy(buf, acc.at[sdv.at[c]], add=True)

    for h in range(2):
        pltpu.sync_copy(sd3.at[wid, pl.ds(h * NCH2, NCH2)], sdv)
        pltpu.sync_copy(ew2.at[wid, pl.ds(h * NCH2 * CK, NCH2 * CK)], ewv)
        _unpack_sd(sdv, srcv, NCH2)

        fire(0, ga, sema)
        fire(1, gb, semb)

        def step(t, _):
            process(2 * t, ga, sema)
            fire(2 * t + 2, ga, sema)
            process(2 * t + 1, gb, semb)
            fire(2 * t + 3, gb, semb)
            return 0

        lax.fori_loop(0, NCH2 // 2 - 1, step, 0)
        process(NCH2 - 2, ga, sema)
        process(NCH2 - 1, gb, semb)

    plsc.subcore_barrier()
    pltpu.sync_copy(acc.at[pl.ds(r0, RT)], out.at[cid, pl.ds(r0, RT)])


@functools.partial(
    pl.kernel,
    out_type=jax.ShapeDtypeStruct((NC, NA, CH), jnp.float32),
    mesh=_mesh,
    scratch_types=[
        pltpu.VMEM((NCH2, CK), jnp.int32),
        pltpu.VMEM((NCH2, CK), jnp.int32),
        pltpu.VMEM((NCH2 * CK,), jnp.float32),
        pltpu.VMEM((CK, CH), jnp.float32),
        pltpu.VMEM((CK, CH), jnp.float32),
        pltpu.VMEM_SHARED((NA, CH), jnp.float32),
        pltpu.SemaphoreType.DMA,
        pltpu.SemaphoreType.DMA,
    ],
    compiler_params=pltpu.CompilerParams(needs_layout_passes=False),
)
def _prop_kernel(xs, sd3, ew2, out, sdv, srcv, ewv, ga, gb, acc, sema, semb):
    _prop_body(xs, sd3, ew2, out, sdv, srcv, ewv, ga, gb, acc, sema, semb)


def kernel(x, edge_index, edge_weight, W1, b1, W2, b2):
    src = edge_index[0].astype(jnp.int32)
    dst = edge_index[1].astype(jnp.int32)
    ew = edge_weight.astype(jnp.float32)
    npad = E_PAD - src.shape[0]
    sd = jnp.concatenate([(dst << 16) | src, jnp.zeros((npad,), jnp.int32)])
    sd3 = sd.reshape(NW, NCH, CK)
    ew2 = jnp.concatenate([ew, jnp.zeros((npad,), jnp.float32)]).reshape(NW, EP)
    b1r = b1.reshape(1, CH)
    b2r = b2.reshape(1, CH)

    dp = _deg_kernel(sd3, ew2).reshape(NW, NP).T
    xs1 = _stage_a(x, W1, dp)
    p = _prop_kernel(xs1, sd3, ew2)
    xs2 = _stage_b(p[0], p[1], xs1, W2, b1r, dp)
    q = _prop_kernel(xs2, sd3, ew2)
    return _stage_c(q[0], q[1], xs2, b2r, dp)
